# baseline (speedup 1.0000x reference)
import functools

import jax
import jax.numpy as jnp
import numpy as np
from jax import lax
from jax.experimental import pallas as pl
from jax.experimental.pallas import tpu as pltpu
from jax.experimental.pallas import tpu_sc as plsc

_VOCAB = 1000000
_EMB = 64
_BATCH = 16
_SEQ = 2048
_N = _BATCH * _SEQ

_NC, _NS, _L = 2, 16, 16
_NW = _NC * _NS
_PER_W = _N // _NW
_C = 128
_NCHUNK = _PER_W // _C


def _pos_table(emb, seq):
    enc = np.zeros((seq, emb), dtype=np.float32)
    pos = np.arange(0.0, seq, dtype=np.float32)[:, None]
    i2 = np.arange(0, emb, 2).astype(np.float32)
    enc[:, 0::2] = np.sin(pos / 10000 ** (i2 / emb))
    enc[:, 1::2] = np.cos(pos / 10000 ** (i2 / emb))
    return enc


_POS = _pos_table(_EMB, _SEQ)


def _sc_body(table_hbm, x_hbm, pos_hbm, out_hbm, idx_v, rows_v, pos_v, sem):
    wid = lax.axis_index("s") * _NC + lax.axis_index("c")
    base_w = wid * _PER_W
    t0 = (wid % 2) * _PER_W

    for c in range(_NCHUNK):
        base = base_w + c * _C
        pltpu.sync_copy(x_hbm.at[pl.ds(base, _C)], idx_v)
        gather = pltpu.async_copy(table_hbm.at[idx_v], rows_v, sem)
        pltpu.sync_copy(pos_hbm.at[pl.ds(t0 + c * _C, _C)], pos_v)
        gather.wait()

        @pl.loop(0, _C)
        def _add(j):
            for k in range(_EMB // _L):
                sl = pl.ds(k * _L, _L)
                rows_v[j, sl] = rows_v[j, sl] + pos_v[j, sl]

        pltpu.sync_copy(rows_v, out_hbm.at[pl.ds(base, _C)])


@jax.jit
def _pos_ntok(x_flat, table):
    mesh = plsc.VectorSubcoreMesh(core_axis_name="c", subcore_axis_name="s")
    fn = pl.kernel(
        _sc_body,
        out_type=jax.ShapeDtypeStruct((_N, _EMB), jnp.float32),
        mesh=mesh,
        scratch_types=[
            pltpu.VMEM((_C,), jnp.int32),
            pltpu.VMEM((_C, _EMB), jnp.float32),
            pltpu.VMEM((_C, _EMB), jnp.float32),
            pltpu.SemaphoreType.DMA,
        ],
        compiler_params=pltpu.CompilerParams(use_tc_tiling_on_sc=False),
    )
    return fn(table, x_flat, jnp.asarray(_POS))


def kernel(x, table):
    out = _pos_ntok(x.reshape(_N), table)
    return out.reshape(_BATCH, _SEQ, _EMB)


# speedup vs baseline: 1.6337x; 1.6337x over previous
import jax
import jax.numpy as jnp
import numpy as np
from jax import lax
from jax.experimental import pallas as pl
from jax.experimental.pallas import tpu as pltpu
from jax.experimental.pallas import tpu_sc as plsc

_VOCAB = 1000000
_EMB = 64
_BATCH = 16
_SEQ = 2048

_NC, _NS, _L = 2, 16, 16
_NW = _NC * _NS
_PER_W = _BATCH * _SEQ // _NW
_C = 128
_NCHUNK = _PER_W // _C


def _pos_table(emb, seq):
    enc = np.zeros((seq, emb), dtype=np.float32)
    pos = np.arange(0.0, seq, dtype=np.float32)[:, None]
    i2 = np.arange(0, emb, 2).astype(np.float32)
    enc[:, 0::2] = np.sin(pos / 10000 ** (i2 / emb))
    enc[:, 1::2] = np.cos(pos / 10000 ** (i2 / emb))
    return enc


_POS = _pos_table(_EMB, _SEQ)


def _sc_body(table_hbm, x_hbm, pos_hbm, out_hbm, idx_v, rows_v, pos_v, sem):
    wid = lax.axis_index("s") * _NC + lax.axis_index("c")
    b = wid // 2
    t_half = (wid % 2) * _PER_W

    @pl.loop(0, _NCHUNK)
    def _chunk(c):
        t = t_half + c * _C
        pltpu.sync_copy(x_hbm.at[b, pl.ds(t, _C)], idx_v)
        pos_cp = pltpu.async_copy(pos_hbm.at[pl.ds(t, _C), :], pos_v, sem)

        @pl.loop(0, _C // _L)
        def _fire(g):
            rv = idx_v[pl.ds(g * _L, _L)]
            for j in range(_L):
                i = g * _L + j
                pltpu.async_copy(table_hbm.at[rv[j], :], rows_v.at[i], sem)

        pos_cp.wait()

        @pl.loop(0, _C)
        def _drain_rows(i):
            pltpu.make_async_copy(table_hbm.at[0, :], rows_v.at[i], sem).wait()

        @pl.loop(0, _C)
        def _add(i):
            for k in range(_EMB // _L):
                sl = pl.ds(k * _L, _L)
                rows_v[i, sl] = rows_v[i, sl] + pos_v[i, sl]

        pltpu.sync_copy(rows_v, out_hbm.at[b, pl.ds(t, _C), :])


@jax.jit
def _pos_ntok(x, table):
    mesh = plsc.VectorSubcoreMesh(core_axis_name="c", subcore_axis_name="s")
    fn = pl.kernel(
        _sc_body,
        out_type=jax.ShapeDtypeStruct((_BATCH, _SEQ, _EMB), jnp.float32),
        mesh=mesh,
        scratch_types=[
            pltpu.VMEM((_C,), jnp.int32),
            pltpu.VMEM((_C, _EMB), jnp.float32),
            pltpu.VMEM((_C, _EMB), jnp.float32),
            pltpu.SemaphoreType.DMA,
        ],
    )
    return fn(table, x, jnp.asarray(_POS))


def kernel(x, table):
    return _pos_ntok(x, table)
